# baseline (speedup 1.0000x reference)
import functools

import jax
import jax.numpy as jnp
from jax.experimental import pallas as pl
from jax.experimental.pallas import tpu as pltpu

_EPS = 1e-5
_VMEM = 58 * 1024 * 1024

_DT = jnp.bfloat16


def _cparams():
    return pltpu.CompilerParams(
        dimension_semantics=("parallel",),
        vmem_limit_bytes=_VMEM,
    )


def _conv_frame(a, w_ref, b_ref, c, wpad):
    pe = a.shape[1]
    xm = pltpu.roll(a, 1, 1)
    xp = pltpu.roll(a, pe - 1, 1)
    x3 = jnp.concatenate([xm, a, xp], axis=0)
    z = jnp.dot(w_ref[...], x3, preferred_element_type=jnp.float32)
    return (pltpu.roll(z[0:c], wpad, 1) + z[c:2 * c]
            + pltpu.roll(z[2 * c:3 * c], pe - wpad, 1) + b_ref[...])


def _stats(acc, mk, s_ref, q_ref):
    m = acc * mk
    s_ref[...] = jnp.sum(m, axis=1, keepdims=True)
    q_ref[...] = jnp.sum(m * m, axis=1, keepdims=True)


def _s1_kernel(x_ref, mk_ref, w_ref, b_ref, y_ref, s_ref, q_ref, xs_ref,
               *, c, h, w, wpad):
    xs_ref[...] = jnp.zeros(xs_ref.shape, xs_ref.dtype)
    for i in range(h):
        xs_ref[:, (i + 1) * wpad + 1:(i + 1) * wpad + 1 + w] = x_ref[:, i, :]
    acc = _conv_frame(xs_ref[...], w_ref, b_ref, c, wpad)
    _stats(acc, mk_ref[...], s_ref, q_ref)
    y_ref[...] = acc.astype(y_ref.dtype)


def _s2_kernel(y_ref, sc_ref, sh_ref, mk_ref, w_ref, b_ref,
               y2_ref, s_ref, q_ref, *, c, wpad):
    mk = mk_ref[...]
    yv = y_ref[...].astype(jnp.float32)
    a = jnp.maximum(yv * sc_ref[...] + sh_ref[...], 0.0) * mk
    acc = _conv_frame(a, w_ref, b_ref, c, wpad)
    _stats(acc, mk, s_ref, q_ref)
    y2_ref[...] = acc.astype(y2_ref.dtype)


def _s3_kernel(y_ref, sc_ref, sh_ref, mk_ref, w_ref, b_ref, o_ref,
               *, c, h, w, wpad):
    mk = mk_ref[...]
    yv = y_ref[...].astype(jnp.float32)
    a = jnp.maximum(yv * sc_ref[...] + sh_ref[...], 0.0) * mk
    acc = _conv_frame(a, w_ref, b_ref, c, wpad)
    res = jnp.maximum(acc + a, 0.0)
    for i in range(h):
        o_ref[:, i, :] = res[:, (i + 1) * wpad + 1:(i + 1) * wpad + 1 + w]


def _affine(s_parts, q_parts, count, gamma, beta):
    s = jnp.sum(s_parts, axis=0)[:, 0]
    q = jnp.sum(q_parts, axis=0)[:, 0]
    mean = s / count
    var = jnp.maximum(q / count - mean * mean, 0.0)
    scale = gamma / jnp.sqrt(var + _EPS)
    shift = beta - mean * scale
    return scale.reshape(-1, 1), shift.reshape(-1, 1)


def _frame_mask(pe, hpad, wpad):
    p = jnp.arange(pe, dtype=jnp.int32)[None, :]
    rp = p // wpad
    cp = p % wpad
    keep = ((p < hpad * wpad) & (rp >= 1) & (rp <= hpad - 2)
            & (cp >= 1) & (cp <= wpad - 2))
    return keep.astype(jnp.float32)


def kernel(x, w1, b1, w2, b2, w3, b3, g1, be1, g2, be2):
    x = x.astype(jnp.float32)
    n, c, h, w = x.shape
    hpad, wpad = h + 2, w + 2
    frame = hpad * wpad
    pe = -(-frame // 128) * 128
    if pe - frame < wpad + 1:
        pe += 128

    wl1 = jnp.transpose(w1, (2, 0, 3, 1)).reshape(3 * c, 3 * c)
    wl2 = jnp.transpose(w2, (2, 0, 3, 1)).reshape(3 * c, 3 * c)
    wl3 = jnp.transpose(w3, (2, 0, 3, 1)).reshape(3 * c, 3 * c)
    bb1 = b1.reshape(c, 1)
    bb2 = b2.reshape(c, 1)
    bb3 = b3.reshape(c, 1)
    mask = _frame_mask(pe, hpad, wpad)

    act_spec = pl.BlockSpec((None, c, pe), lambda i: (i, 0, 0))
    nat_spec = pl.BlockSpec((None, c, h, w), lambda i: (i, 0, 0, 0))
    w_spec = pl.BlockSpec((3 * c, 3 * c), lambda i: (0, 0))
    col_spec = pl.BlockSpec((c, 1), lambda i: (0, 0))
    mask_spec = pl.BlockSpec((1, pe), lambda i: (0, 0))
    stat_spec = pl.BlockSpec((None, c, 1), lambda i: (i, 0, 0))
    stat_shape = jax.ShapeDtypeStruct((n, c, 1), jnp.float32)

    y1, s1, q1 = pl.pallas_call(
        functools.partial(_s1_kernel, c=c, h=h, w=w, wpad=wpad),
        out_shape=(jax.ShapeDtypeStruct((n, c, pe), _DT),
                   stat_shape, stat_shape),
        grid=(n,),
        in_specs=[nat_spec, mask_spec, w_spec, col_spec],
        out_specs=(act_spec, stat_spec, stat_spec),
        scratch_shapes=[pltpu.VMEM((c, pe), jnp.float32)],
        compiler_params=_cparams(),
    )(x, mask, wl1, bb1)

    sc1, sh1 = _affine(s1, q1, n * h * w, g1, be1)

    y2, s2, q2 = pl.pallas_call(
        functools.partial(_s2_kernel, c=c, wpad=wpad),
        out_shape=(jax.ShapeDtypeStruct((n, c, pe), _DT),
                   stat_shape, stat_shape),
        grid=(n,),
        in_specs=[act_spec, col_spec, col_spec, mask_spec, w_spec, col_spec],
        out_specs=(act_spec, stat_spec, stat_spec),
        compiler_params=_cparams(),
    )(y1, sc1, sh1, mask, wl2, bb2)

    sc2, sh2 = _affine(s2, q2, n * h * w, g2, be2)

    out = pl.pallas_call(
        functools.partial(_s3_kernel, c=c, h=h, w=w, wpad=wpad),
        out_shape=jax.ShapeDtypeStruct((n, c, h, w), jnp.float32),
        grid=(n,),
        in_specs=[act_spec, col_spec, col_spec, mask_spec, w_spec, col_spec],
        out_specs=nat_spec,
        compiler_params=_cparams(),
    )(y2, sc2, sh2, mask, wl3, bb3)

    return out


# speedup vs baseline: 1.9989x; 1.9989x over previous
import functools

import jax
import jax.numpy as jnp
from jax import lax
from jax.experimental import pallas as pl
from jax.experimental.pallas import tpu as pltpu

_EPS = 1e-5
_VMEM = 48 * 1024 * 1024

_DT = jnp.bfloat16


def _cparams():
    return pltpu.CompilerParams(
        dimension_semantics=("parallel",),
        vmem_limit_bytes=_VMEM,
    )


def _pack_weights(w_oihw, dt):
    t = jnp.transpose(w_oihw, (2, 3, 1, 0))
    z = jnp.zeros_like(t[0, 0])
    mats = []
    for kh in range(3):
        t0, t1, t2 = t[kh, 0], t[kh, 1], t[kh, 2]
        w_a = jnp.concatenate(
            [jnp.concatenate([t0, z], axis=1),
             jnp.concatenate([t1, t0], axis=1)], axis=0)
        w_b = jnp.concatenate(
            [jnp.concatenate([t2, t1], axis=1),
             jnp.concatenate([z, t2], axis=1)], axis=0)
        mats += [w_a, w_b]
    return jnp.stack(mats).astype(dt)


def _conv6(a, w_ref, wh, ldh):
    acc = jnp.dot(a[0:ldh, :], w_ref[0], preferred_element_type=jnp.float32)
    for s in range(1, 6):
        kh, sh = divmod(s, 2)
        off = kh * wh + sh
        acc = acc + jnp.dot(a[off:off + ldh, :], w_ref[s],
                            preferred_element_type=jnp.float32)
    return acc


def _stats(acc, wh, ldh, s_ref, q_ref):
    r = lax.broadcasted_iota(jnp.int32, (ldh, 1), 0)
    m = jnp.where((r % wh) < (wh - 1), acc, 0.0)
    s_ref[...] = jnp.sum(m, axis=0, keepdims=True)
    q_ref[...] = jnp.sum(m * m, axis=0, keepdims=True)


def _store_packed(y_ref, acc, wh, ldh):
    y_ref[...] = jnp.zeros(y_ref.shape, y_ref.dtype)
    y_ref[wh:wh + ldh, 128:] = acc[:, :128].astype(y_ref.dtype)
    y_ref[wh + 1:wh + 1 + ldh, :128] = acc[:, 128:].astype(y_ref.dtype)


def _s1_kernel(x_ref, w_ref, b_ref, y_ref, s_ref, q_ref, *, wh, ldh):
    acc = _conv6(x_ref[...].astype(jnp.float32), w_ref, wh, ldh) + b_ref[...]
    _stats(acc, wh, ldh, s_ref, q_ref)
    _store_packed(y_ref, acc, wh, ldh)


def _s2_kernel(y_ref, sc_ref, sh_ref, mk_ref, w_ref, b_ref,
               y2_ref, s_ref, q_ref, *, wh, ldh):
    yv = y_ref[...].astype(jnp.float32)
    a = jnp.maximum(yv * sc_ref[...] + sh_ref[...], 0.0) * mk_ref[...]
    acc = _conv6(a, w_ref, wh, ldh) + b_ref[...]
    _stats(acc, wh, ldh, s_ref, q_ref)
    _store_packed(y2_ref, acc, wh, ldh)


def _s3_kernel(y_ref, sc_ref, sh_ref, mk_ref, w_ref, b_ref, o_ref,
               *, wh, ldh):
    yv = y_ref[...].astype(jnp.float32)
    a = jnp.maximum(yv * sc_ref[...] + sh_ref[...], 0.0) * mk_ref[...]
    acc = _conv6(a, w_ref, wh, ldh) + b_ref[...]
    res = jnp.concatenate(
        [a[wh:wh + ldh, 128:], a[wh + 1:wh + 1 + ldh, :128]], axis=1)
    o_ref[...] = jnp.maximum(acc + res, 0.0)


def _affine(s_parts, q_parts, count, gamma, beta):
    s = jnp.sum(s_parts, axis=0)[0]
    q = jnp.sum(q_parts, axis=0)[0]
    s = s[:128] + s[128:]
    q = q[:128] + q[128:]
    mean = s / count
    var = jnp.maximum(q / count - mean * mean, 0.0)
    scale = gamma / jnp.sqrt(var + _EPS)
    shift = beta - mean * scale
    sc2 = jnp.concatenate([scale, scale]).reshape(1, 256)
    sh2 = jnp.concatenate([shift, shift]).reshape(1, 256)
    return sc2.astype(jnp.float32), sh2.astype(jnp.float32)


def _interior_mask(pe2, hpad, wpad):
    rows = jnp.arange(pe2, dtype=jnp.int32)[:, None]
    lanes = jnp.arange(256, dtype=jnp.int32)[None, :]
    p = 2 * rows + (lanes >= 128).astype(jnp.int32)
    rp = p // wpad
    cp = p % wpad
    keep = ((p < hpad * wpad) & (rp >= 1) & (rp <= hpad - 2)
            & (cp >= 1) & (cp <= wpad - 2))
    return keep.astype(jnp.float32)


def _dup(v):
    return jnp.concatenate([v, v]).reshape(1, 256).astype(jnp.float32)


def kernel(x, w1, b1, w2, b2, w3, b3, g1, be1, g2, be2):
    x = x.astype(jnp.float32)
    n, c, h, w = x.shape
    hpad, wpad = h + 2, w + 2
    wh = wpad // 2
    ldh = h * wh
    half = hpad * wpad // 2
    pe2 = -(-(ldh + 2 * wh + 1) // 16) * 16
    pe2 = max(pe2, -(-half // 16) * 16)

    xt = jnp.transpose(x, (0, 2, 3, 1))
    xp = jnp.pad(xt, ((0, 0), (1, 1), (1, 1), (0, 0)))
    x2 = xp.reshape(n, half, 2 * c)
    x2 = jnp.pad(x2, ((0, 0), (0, pe2 - half), (0, 0))).astype(_DT)

    wp1 = _pack_weights(w1, jnp.float32)
    wp2 = _pack_weights(w2, jnp.float32)
    wp3 = _pack_weights(w3, jnp.float32)
    bb1, bb2, bb3 = _dup(b1), _dup(b2), _dup(b3)
    mask = _interior_mask(pe2, hpad, wpad)

    act_spec = pl.BlockSpec((None, pe2, 256), lambda i: (i, 0, 0))
    w_spec = pl.BlockSpec((6, 256, 256), lambda i: (0, 0, 0))
    vec_spec = pl.BlockSpec((1, 256), lambda i: (0, 0))
    mask_spec = pl.BlockSpec((pe2, 256), lambda i: (0, 0))
    stat_spec = pl.BlockSpec((None, 1, 256), lambda i: (i, 0, 0))

    y1, s1, q1 = pl.pallas_call(
        functools.partial(_s1_kernel, wh=wh, ldh=ldh),
        out_shape=(
            jax.ShapeDtypeStruct((n, pe2, 256), _DT),
            jax.ShapeDtypeStruct((n, 1, 256), jnp.float32),
            jax.ShapeDtypeStruct((n, 1, 256), jnp.float32),
        ),
        grid=(n,),
        in_specs=[act_spec, w_spec, vec_spec],
        out_specs=(act_spec, stat_spec, stat_spec),
        compiler_params=_cparams(),
    )(x2, wp1, bb1)

    sc1, sh1 = _affine(s1, q1, n * h * w, g1, be1)

    y2, s2, q2 = pl.pallas_call(
        functools.partial(_s2_kernel, wh=wh, ldh=ldh),
        out_shape=(
            jax.ShapeDtypeStruct((n, pe2, 256), _DT),
            jax.ShapeDtypeStruct((n, 1, 256), jnp.float32),
            jax.ShapeDtypeStruct((n, 1, 256), jnp.float32),
        ),
        grid=(n,),
        in_specs=[act_spec, vec_spec, vec_spec, mask_spec, w_spec, vec_spec],
        out_specs=(act_spec, stat_spec, stat_spec),
        compiler_params=_cparams(),
    )(y1, sc1, sh1, mask, wp2, bb2)

    sc2, sh2 = _affine(s2, q2, n * h * w, g2, be2)

    out = pl.pallas_call(
        functools.partial(_s3_kernel, wh=wh, ldh=ldh),
        out_shape=jax.ShapeDtypeStruct((n, ldh, 256), jnp.float32),
        grid=(n,),
        in_specs=[act_spec, vec_spec, vec_spec, mask_spec, w_spec, vec_spec],
        out_specs=pl.BlockSpec((None, ldh, 256), lambda i: (i, 0, 0)),
        compiler_params=_cparams(),
    )(y2, sc2, sh2, mask, wp3, bb3)

    out = out.reshape(n, h, wh, 2, c).reshape(n, h, wpad, c)[:, :, :w, :]
    return jnp.transpose(out, (0, 3, 1, 2))


# speedup vs baseline: 2.0900x; 1.0456x over previous
import functools

import jax
import jax.numpy as jnp
from jax import lax
from jax.experimental import pallas as pl
from jax.experimental.pallas import tpu as pltpu

_EPS = 1e-5
_VMEM = 48 * 1024 * 1024

_DT = jnp.bfloat16


def _cparams():
    return pltpu.CompilerParams(
        dimension_semantics=("parallel",),
        vmem_limit_bytes=_VMEM,
    )


def _pack_weights(w_oihw, dt):
    t = jnp.transpose(w_oihw, (2, 3, 1, 0))
    z = jnp.zeros_like(t[0, 0])
    mats = []
    for kh in range(3):
        t0, t1, t2 = t[kh, 0], t[kh, 1], t[kh, 2]
        w_a = jnp.concatenate(
            [jnp.concatenate([t0, z], axis=1),
             jnp.concatenate([t1, t0], axis=1)], axis=0)
        w_b = jnp.concatenate(
            [jnp.concatenate([t2, t1], axis=1),
             jnp.concatenate([z, t2], axis=1)], axis=0)
        mats += [w_a, w_b]
    return jnp.stack(mats).astype(dt)


def _conv6(a, w_ref, wh, ldh):
    acc = jnp.dot(a[0:ldh, :], w_ref[0], preferred_element_type=jnp.float32)
    for s in range(1, 6):
        kh, sh = divmod(s, 2)
        off = kh * wh + sh
        acc = acc + jnp.dot(a[off:off + ldh, :], w_ref[s],
                            preferred_element_type=jnp.float32)
    return acc


def _stats(acc, wh, ldh, s_ref, q_ref):
    r = lax.broadcasted_iota(jnp.int32, (ldh, 1), 0)
    m = jnp.where((r % wh) < (wh - 1), acc, 0.0)
    s_ref[...] = jnp.sum(m, axis=0, keepdims=True)
    q_ref[...] = jnp.sum(m * m, axis=0, keepdims=True)


def _store_packed(y_ref, acc, wh, ldh):
    y_ref[...] = jnp.zeros(y_ref.shape, y_ref.dtype)
    y_ref[wh:wh + ldh, 128:] = acc[:, :128].astype(y_ref.dtype)
    y_ref[wh + 1:wh + 1 + ldh, :128] = acc[:, 128:].astype(y_ref.dtype)


def _s1_kernel(x_ref, w_ref, b_ref, y_ref, s_ref, q_ref, *, wh, ldh):
    acc = _conv6(x_ref[...].astype(jnp.float32), w_ref, wh, ldh) + b_ref[...]
    _stats(acc, wh, ldh, s_ref, q_ref)
    _store_packed(y_ref, acc, wh, ldh)


def _s2_kernel(y_ref, sc_ref, sh_ref, mk_ref, w_ref, b_ref,
               y2_ref, s_ref, q_ref, *, wh, ldh):
    yv = y_ref[...].astype(jnp.float32)
    a = jnp.maximum(yv * sc_ref[...] + sh_ref[...], 0.0) * mk_ref[...]
    acc = _conv6(a, w_ref, wh, ldh) + b_ref[...]
    _stats(acc, wh, ldh, s_ref, q_ref)
    _store_packed(y2_ref, acc, wh, ldh)


def _s3_kernel(y_ref, sc_ref, sh_ref, mk_ref, w_ref, b_ref, o_ref,
               *, wh, ldh):
    yv = y_ref[...].astype(jnp.float32)
    a = jnp.maximum(yv * sc_ref[...] + sh_ref[...], 0.0) * mk_ref[...]
    acc = _conv6(a, w_ref, wh, ldh) + b_ref[...]
    res = jnp.concatenate(
        [a[wh:wh + ldh, 128:], a[wh + 1:wh + 1 + ldh, :128]], axis=1)
    o_ref[...] = jnp.maximum(acc + res, 0.0).astype(o_ref.dtype)


def _affine(s_parts, q_parts, count, gamma, beta):
    s = jnp.sum(s_parts, axis=0)[0]
    q = jnp.sum(q_parts, axis=0)[0]
    s = s[:128] + s[128:]
    q = q[:128] + q[128:]
    mean = s / count
    var = jnp.maximum(q / count - mean * mean, 0.0)
    scale = gamma / jnp.sqrt(var + _EPS)
    shift = beta - mean * scale
    sc2 = jnp.concatenate([scale, scale]).reshape(1, 256)
    sh2 = jnp.concatenate([shift, shift]).reshape(1, 256)
    return sc2.astype(jnp.float32), sh2.astype(jnp.float32)


def _interior_mask(pe2, hpad, wpad):
    rows = jnp.arange(pe2, dtype=jnp.int32)[:, None]
    lanes = jnp.arange(256, dtype=jnp.int32)[None, :]
    p = 2 * rows + (lanes >= 128).astype(jnp.int32)
    rp = p // wpad
    cp = p % wpad
    keep = ((p < hpad * wpad) & (rp >= 1) & (rp <= hpad - 2)
            & (cp >= 1) & (cp <= wpad - 2))
    return keep.astype(jnp.float32)


def _dup(v):
    return jnp.concatenate([v, v]).reshape(1, 256).astype(jnp.float32)


def kernel(x, w1, b1, w2, b2, w3, b3, g1, be1, g2, be2):
    x = x.astype(jnp.float32)
    n, c, h, w = x.shape
    hpad, wpad = h + 2, w + 2
    wh = wpad // 2
    ldh = h * wh
    half = hpad * wpad // 2
    pe2 = -(-(ldh + 2 * wh + 1) // 16) * 16
    pe2 = max(pe2, -(-half // 16) * 16)

    xt = jnp.transpose(x, (0, 2, 3, 1))
    xp = jnp.pad(xt, ((0, 0), (1, 1), (1, 1), (0, 0)))
    x2 = xp.reshape(n, half, 2 * c)
    x2 = jnp.pad(x2, ((0, 0), (0, pe2 - half), (0, 0))).astype(_DT)

    wp1 = _pack_weights(w1, jnp.float32)
    wp2 = _pack_weights(w2, jnp.float32)
    wp3 = _pack_weights(w3, jnp.float32)
    bb1, bb2, bb3 = _dup(b1), _dup(b2), _dup(b3)
    mask = _interior_mask(pe2, hpad, wpad)

    act_spec = pl.BlockSpec((None, pe2, 256), lambda i: (i, 0, 0))
    w_spec = pl.BlockSpec((6, 256, 256), lambda i: (0, 0, 0))
    vec_spec = pl.BlockSpec((1, 256), lambda i: (0, 0))
    mask_spec = pl.BlockSpec((pe2, 256), lambda i: (0, 0))
    stat_spec = pl.BlockSpec((None, 1, 256), lambda i: (i, 0, 0))

    y1, s1, q1 = pl.pallas_call(
        functools.partial(_s1_kernel, wh=wh, ldh=ldh),
        out_shape=(
            jax.ShapeDtypeStruct((n, pe2, 256), _DT),
            jax.ShapeDtypeStruct((n, 1, 256), jnp.float32),
            jax.ShapeDtypeStruct((n, 1, 256), jnp.float32),
        ),
        grid=(n,),
        in_specs=[act_spec, w_spec, vec_spec],
        out_specs=(act_spec, stat_spec, stat_spec),
        compiler_params=_cparams(),
    )(x2, wp1, bb1)

    sc1, sh1 = _affine(s1, q1, n * h * w, g1, be1)

    y2, s2, q2 = pl.pallas_call(
        functools.partial(_s2_kernel, wh=wh, ldh=ldh),
        out_shape=(
            jax.ShapeDtypeStruct((n, pe2, 256), _DT),
            jax.ShapeDtypeStruct((n, 1, 256), jnp.float32),
            jax.ShapeDtypeStruct((n, 1, 256), jnp.float32),
        ),
        grid=(n,),
        in_specs=[act_spec, vec_spec, vec_spec, mask_spec, w_spec, vec_spec],
        out_specs=(act_spec, stat_spec, stat_spec),
        compiler_params=_cparams(),
    )(y1, sc1, sh1, mask, wp2, bb2)

    sc2, sh2 = _affine(s2, q2, n * h * w, g2, be2)

    out = pl.pallas_call(
        functools.partial(_s3_kernel, wh=wh, ldh=ldh),
        out_shape=jax.ShapeDtypeStruct((n, ldh, 256), _DT),
        grid=(n,),
        in_specs=[act_spec, vec_spec, vec_spec, mask_spec, w_spec, vec_spec],
        out_specs=pl.BlockSpec((None, ldh, 256), lambda i: (i, 0, 0)),
        compiler_params=_cparams(),
    )(y2, sc2, sh2, mask, wp3, bb3)

    out = out.reshape(n, h, wh, 2, c).reshape(n, h, wpad, c)[:, :, :w, :]
    return jnp.transpose(out, (0, 3, 1, 2)).astype(jnp.float32)
